# baseline (speedup 1.0000x reference)
import functools

import jax
import jax.numpy as jnp
from jax import lax
from jax.experimental import pallas as pl
from jax.experimental.pallas import tpu as pltpu
from jax.experimental.pallas import tpu_sc as plsc

_NC = 2
_NS = 16
_NW = _NC * _NS
_L = 16


def _sc_body(comb_hbm, out_hbm, comb_v, chunk_v, sem, *, b, h, w, d, ch_per_w):
    hw = h * w
    cid = lax.axis_index("c")
    sid = lax.axis_index("s")
    wid = sid * _NC + cid

    pltpu.sync_copy(comb_hbm, comb_v)

    base_c = wid * ch_per_w
    lane = lax.iota(jnp.int32, _L)
    is_col = wid * ch_per_w < d

    groups_per_ch = hw // _L

    def body(n, _):
        t = n // groups_per_ch
        g = n - t * groups_per_ch
        c = base_c + t
        k = g * _L + lane
        idx_col = (k % w) * d + c
        idx_row = w * d + (k // w) * d + (c - d)
        idx = jnp.where(is_col, idx_col, idx_row)
        vals = plsc.load_gather(comb_v, [idx])
        chunk_v[pl.ds(n * _L, _L)] = vals
        return 0

    lax.fori_loop(0, ch_per_w * groups_per_ch, body, 0)

    descs = [pltpu.async_copy(chunk_v, out_hbm.at[i, wid], sem)
             for i in range(b)]
    for dsc in descs:
        dsc.wait()


def kernel(x, row_weight, col_weight):
    b = x.shape[0]
    h, w = x.shape[-2], x.shape[-1]
    d = row_weight.shape[1]
    hw = h * w
    ch_per_w = 2 * d // _NW
    chunk = ch_per_w * hw

    comb = jnp.concatenate(
        [col_weight[:w].reshape(-1), row_weight[:h].reshape(-1)])

    mesh = plsc.VectorSubcoreMesh(core_axis_name="c", subcore_axis_name="s")
    body = functools.partial(
        _sc_body, b=b, h=h, w=w, d=d, ch_per_w=ch_per_w)
    run = pl.kernel(
        body,
        out_type=jax.ShapeDtypeStruct((b, _NW, chunk), jnp.float32),
        mesh=mesh,
        scratch_types=[
            pltpu.VMEM((comb.shape[0],), jnp.float32),
            pltpu.VMEM((chunk,), jnp.float32),
            pltpu.SemaphoreType.DMA,
        ],
        compiler_params=pltpu.CompilerParams(needs_layout_passes=False),
    )
    out = run(comb)
    return out.reshape(b, 2 * d, h, w)


# speedup vs baseline: 5.2414x; 5.2414x over previous
import functools

import jax
import jax.numpy as jnp
from jax import lax
from jax.experimental import pallas as pl
from jax.experimental.pallas import tpu as pltpu
from jax.experimental.pallas import tpu_sc as plsc

_NC = 2
_NS = 16
_NW = _NC * _NS
_L = 16


def _sc_body(comb_hbm, out_hbm, comb_v, chunk_v, sem, *, b, h, w, d):
    hw = h * w
    px_per_w = hw // _NW
    gr_per_px = 2 * d // _L
    gr_col = d // _L

    cid = lax.axis_index("c")
    sid = lax.axis_index("s")
    wid = sid * _NC + cid

    pltpu.sync_copy(comb_hbm, comb_v)

    base_px = wid * px_per_w

    def body(n, _):
        lp = n // gr_per_px
        cg = n - lp * gr_per_px
        p = base_px + lp
        i = p // w
        j = p - i * w
        src = jnp.where(cg < gr_col,
                        j * d + cg * _L,
                        w * d + i * d + (cg - gr_col) * _L)
        chunk_v[pl.ds(n * _L, _L)] = comb_v[pl.ds(src, _L)]
        return 0

    lax.fori_loop(0, px_per_w * gr_per_px, body, 0)

    descs = [pltpu.async_copy(chunk_v, out_hbm.at[i, wid], sem)
             for i in range(b)]
    for dsc in descs:
        dsc.wait()


def kernel(x, row_weight, col_weight):
    b = x.shape[0]
    h, w = x.shape[-2], x.shape[-1]
    d = row_weight.shape[1]
    hw = h * w
    chunk = (hw // _NW) * 2 * d

    comb = jnp.concatenate(
        [col_weight[:w].reshape(-1), row_weight[:h].reshape(-1)])

    mesh = plsc.VectorSubcoreMesh(core_axis_name="c", subcore_axis_name="s")
    body = functools.partial(_sc_body, b=b, h=h, w=w, d=d)
    run = pl.kernel(
        body,
        out_type=jax.ShapeDtypeStruct((b, _NW, chunk), jnp.float32),
        mesh=mesh,
        scratch_types=[
            pltpu.VMEM((comb.shape[0],), jnp.float32),
            pltpu.VMEM((chunk,), jnp.float32),
            pltpu.SemaphoreType.DMA,
        ],
        compiler_params=pltpu.CompilerParams(needs_layout_passes=False),
    )
    out = run(comb)
    return out.reshape(b, h, w, 2 * d).transpose(0, 3, 1, 2)


# speedup vs baseline: 9.3841x; 1.7904x over previous
import functools

import jax
import jax.numpy as jnp
from jax import lax
from jax.experimental import pallas as pl
from jax.experimental.pallas import tpu as pltpu
from jax.experimental.pallas import tpu_sc as plsc

_NC = 2
_NS = 16
_NW = _NC * _NS
_L = 16
_SG = 8
_BG = _NW // _SG


def _sc_body(comb_hbm, out_hbm, comb_v, chunk_v, sem, *, b, h, w, d):
    gr = d // _L
    ti = h // _SG
    bn = b // _BG

    cid = lax.axis_index("c")
    sid = lax.axis_index("s")
    wid = sid * _NC + cid
    sg = wid % _SG
    bg = wid // _SG
    i0 = sg * ti
    b0 = bg * bn

    pltpu.sync_copy(comb_hbm, comb_v)

    def body(px, _):
        r = px // w
        j = px - r * w
        for cg in range(gr):
            chunk_v[r, j, pl.ds(cg * _L, _L)] = (
                comb_v[pl.ds(j * d + cg * _L, _L)])
        for cg in range(gr):
            chunk_v[r, j, pl.ds(d + cg * _L, _L)] = (
                comb_v[pl.ds(w * d + (i0 + r) * d + cg * _L, _L)])
        return 0

    lax.fori_loop(0, ti * w, body, 0)

    descs = [
        pltpu.async_copy(chunk_v, out_hbm.at[b0 + k, pl.ds(i0, ti)], sem)
        for k in range(bn)
    ]
    for dsc in descs:
        dsc.wait()


def kernel(x, row_weight, col_weight):
    b = x.shape[0]
    h, w = x.shape[-2], x.shape[-1]
    d = row_weight.shape[1]

    comb = jnp.concatenate(
        [col_weight[:w].reshape(-1), row_weight[:h].reshape(-1)])

    mesh = plsc.VectorSubcoreMesh(core_axis_name="c", subcore_axis_name="s")
    body = functools.partial(_sc_body, b=b, h=h, w=w, d=d)
    run = pl.kernel(
        body,
        out_type=jax.ShapeDtypeStruct((b, h, w, 2 * d), jnp.float32),
        mesh=mesh,
        scratch_types=[
            pltpu.VMEM((comb.shape[0],), jnp.float32),
            pltpu.VMEM((h // _SG, w, 2 * d), jnp.float32),
            pltpu.SemaphoreType.DMA,
        ],
        compiler_params=pltpu.CompilerParams(needs_layout_passes=False),
    )
    out = run(comb)
    return out.transpose(0, 3, 1, 2)


# speedup vs baseline: 10.0568x; 1.0717x over previous
import functools

import jax
import jax.numpy as jnp
from jax import lax
from jax.experimental import pallas as pl
from jax.experimental.pallas import tpu as pltpu
from jax.experimental.pallas import tpu_sc as plsc

_NC = 2
_NS = 16
_NW = _NC * _NS
_L = 16
_SG = 8
_BG = _NW // _SG


def _sc_body(row_hbm, col_hbm, out_hbm, row_v, col_v, chunk_v, sem,
             *, b, h, w, d):
    gr = d // _L
    ti = h // _SG
    bn = b // _BG

    cid = lax.axis_index("c")
    sid = lax.axis_index("s")
    wid = sid * _NC + cid
    sg = wid % _SG
    bg = wid // _SG
    i0 = sg * ti
    b0 = bg * bn

    pltpu.sync_copy(row_hbm.at[pl.ds(0, h)], row_v)
    pltpu.sync_copy(col_hbm.at[pl.ds(0, w)], col_v)

    descs = []
    for r in range(ti):
        def body(j, _, r=r):
            for cg in range(gr):
                chunk_v[r, j, pl.ds(cg * _L, _L)] = (
                    col_v[j, pl.ds(cg * _L, _L)])
            for cg in range(gr):
                chunk_v[r, j, pl.ds(d + cg * _L, _L)] = (
                    row_v[i0 + r, pl.ds(cg * _L, _L)])
            return 0

        lax.fori_loop(0, w, body, 0)
        descs.extend(
            pltpu.async_copy(chunk_v.at[r], out_hbm.at[b0 + k, i0 + r], sem)
            for k in range(bn))
    for dsc in descs:
        dsc.wait()


def kernel(x, row_weight, col_weight):
    b = x.shape[0]
    h, w = x.shape[-2], x.shape[-1]
    d = row_weight.shape[1]

    mesh = plsc.VectorSubcoreMesh(core_axis_name="c", subcore_axis_name="s")
    body = functools.partial(_sc_body, b=b, h=h, w=w, d=d)
    run = pl.kernel(
        body,
        out_type=jax.ShapeDtypeStruct((b, h, w, 2 * d), jnp.float32),
        mesh=mesh,
        scratch_types=[
            pltpu.VMEM((h, d), jnp.float32),
            pltpu.VMEM((w, d), jnp.float32),
            pltpu.VMEM((h // _SG, w, 2 * d), jnp.float32),
            pltpu.SemaphoreType.DMA,
        ],
        compiler_params=pltpu.CompilerParams(needs_layout_passes=False),
    )
    out = run(row_weight, col_weight)
    return out.transpose(0, 3, 1, 2)


# speedup vs baseline: 10.2693x; 1.0211x over previous
import functools

import jax
import jax.numpy as jnp
from jax import lax
from jax.experimental import pallas as pl
from jax.experimental.pallas import tpu as pltpu
from jax.experimental.pallas import tpu_sc as plsc

_NC = 2
_NS = 16
_NW = _NC * _NS
_L = 16
_SG = 8
_BG = _NW // _SG


def _sc_body(row_hbm, col_hbm, out_hbm, row_v, col_v, chunk_v, sem, sem2,
             *, b, h, w, d):
    gr = d // _L
    ti = h // _SG
    bn = b // _BG

    cid = lax.axis_index("c")
    sid = lax.axis_index("s")
    wid = sid * _NC + cid
    sg = wid % _SG
    bg = wid // _SG
    i0 = sg * ti
    b0 = bg * bn

    stage_row = pltpu.async_copy(row_hbm.at[pl.ds(0, h)], row_v, sem2)
    stage_col = pltpu.async_copy(col_hbm.at[pl.ds(0, w)], col_v, sem2)
    stage_row.wait()
    stage_col.wait()

    descs = []
    for r in range(ti):
        def body(j, _, r=r):
            for cg in range(gr):
                chunk_v[r, j, pl.ds(cg * _L, _L)] = (
                    col_v[j, pl.ds(cg * _L, _L)])
            for cg in range(gr):
                chunk_v[r, j, pl.ds(d + cg * _L, _L)] = (
                    row_v[i0 + r, pl.ds(cg * _L, _L)])
            return 0

        lax.fori_loop(0, w, body, 0)
        descs.extend(
            pltpu.async_copy(chunk_v.at[r], out_hbm.at[b0 + k, i0 + r], sem)
            for k in range(bn))
    for dsc in descs:
        dsc.wait()


def kernel(x, row_weight, col_weight):
    b = x.shape[0]
    h, w = x.shape[-2], x.shape[-1]
    d = row_weight.shape[1]

    mesh = plsc.VectorSubcoreMesh(core_axis_name="c", subcore_axis_name="s")
    body = functools.partial(_sc_body, b=b, h=h, w=w, d=d)
    run = pl.kernel(
        body,
        out_type=jax.ShapeDtypeStruct((b, h, w, 2 * d), jnp.float32),
        mesh=mesh,
        scratch_types=[
            pltpu.VMEM((h, d), jnp.float32),
            pltpu.VMEM((w, d), jnp.float32),
            pltpu.VMEM((h // _SG, w, 2 * d), jnp.float32),
            pltpu.SemaphoreType.DMA,
            pltpu.SemaphoreType.DMA,
        ],
        compiler_params=pltpu.CompilerParams(needs_layout_passes=False),
    )
    out = run(row_weight, col_weight)
    return out.transpose(0, 3, 1, 2)


# speedup vs baseline: 10.6939x; 1.0413x over previous
import functools

import jax
import jax.numpy as jnp
from jax import lax
from jax.experimental import pallas as pl
from jax.experimental.pallas import tpu as pltpu
from jax.experimental.pallas import tpu_sc as plsc

_NC = 2
_NS = 16
_NW = _NC * _NS
_L = 16
_SG = 8
_BG = _NW // _SG


def _sc_body(row_hbm, col_hbm, out_hbm, row_v, col_v, chunk_v, sem, sem2,
             *, b, h, w, d):
    gr = d // _L
    ti = h // _SG
    bn = b // _BG

    cid = lax.axis_index("c")
    sid = lax.axis_index("s")
    wid = sid * _NC + cid
    sg = wid % _SG
    bg = wid // _SG
    i0 = sg * ti
    b0 = bg * bn

    stage_row = pltpu.async_copy(row_hbm.at[pl.ds(0, h)], row_v, sem2)
    stage_col = pltpu.async_copy(col_hbm.at[pl.ds(0, w)], col_v, sem2)
    stage_row.wait()
    stage_col.wait()

    descs = []
    for r in range(ti):
        rvals = [row_v[i0 + r, pl.ds(cg * _L, _L)] for cg in range(gr)]

        def body(j, _, r=r, rvals=rvals):
            for cg in range(gr):
                chunk_v[r, j, pl.ds(cg * _L, _L)] = (
                    col_v[j, pl.ds(cg * _L, _L)])
            for cg in range(gr):
                chunk_v[r, j, pl.ds(d + cg * _L, _L)] = rvals[cg]
            return 0

        lax.fori_loop(0, w, body, 0)
        descs.extend(
            pltpu.async_copy(chunk_v.at[r], out_hbm.at[b0 + k, i0 + r], sem)
            for k in range(bn))
    for dsc in descs:
        dsc.wait()


def kernel(x, row_weight, col_weight):
    b = x.shape[0]
    h, w = x.shape[-2], x.shape[-1]
    d = row_weight.shape[1]

    mesh = plsc.VectorSubcoreMesh(core_axis_name="c", subcore_axis_name="s")
    body = functools.partial(_sc_body, b=b, h=h, w=w, d=d)
    run = pl.kernel(
        body,
        out_type=jax.ShapeDtypeStruct((b, h, w, 2 * d), jnp.float32),
        mesh=mesh,
        scratch_types=[
            pltpu.VMEM((h, d), jnp.float32),
            pltpu.VMEM((w, d), jnp.float32),
            pltpu.VMEM((h // _SG, w, 2 * d), jnp.float32),
            pltpu.SemaphoreType.DMA,
            pltpu.SemaphoreType.DMA,
        ],
        compiler_params=pltpu.CompilerParams(needs_layout_passes=False),
    )
    out = run(row_weight, col_weight)
    return out.transpose(0, 3, 1, 2)
